# Initial kernel scaffold; baseline (speedup 1.0000x reference)
import jax
import jax.numpy as jnp
from jax.experimental import pallas as pl


def kernel(inputs, edge_index, edge_weight, W_self1, W_neigh1, b1, W_self2, W_neigh2, b2):
    raise NotImplementedError("write your pallas kernel here")


# speedup vs baseline: 3.2324x; 3.2324x over previous
import functools

import jax
import jax.numpy as jnp
from jax import lax
from jax.experimental import pallas as pl
from jax.experimental.pallas import tpu as pltpu
from jax.experimental.pallas import tpu_sc as plsc

N = 10000
E = 320000
D = 128

NC = 2
NS = 16
NW = NC * NS
K = 128
CHUNKS = 79
EPW = CHUNKS * K
EPAD = EPW * NW

NPAD = 10240
RPS = NPAD // NS
NZB = RPS // K


def _sc_agg_body(g_hbm, src_hbm, dst_hbm, w_hbm, *refs, with_deg):
    if with_deg:
        (s_out, deg_out, acc_sh, rows_v, idx_v, dst_v, w_v, sem) = refs
    else:
        (s_out, acc_sh, rows_v, idx_v, dst_v, w_v, sem) = refs

    c = lax.axis_index("c")
    s = lax.axis_index("s")
    wid = c * NS + s

    zero16 = jnp.zeros((16,), jnp.float32)

    def _zrow(i, _):
        for j in range(D // 16):
            rows_v[i, pl.ds(j * 16, 16)] = zero16
        return 0
    lax.fori_loop(0, K, _zrow, 0)
    for k in range(NZB):
        pltpu.sync_copy(rows_v, acc_sh.at[pl.ds(s * RPS + k * K, K)])

    plsc.subcore_barrier()

    def _chunk(t, _):
        base = wid * EPW + t * K
        pltpu.sync_copy(src_hbm.at[pl.ds(base, K)], idx_v)
        pltpu.sync_copy(dst_hbm.at[pl.ds(base, K)], dst_v)
        pltpu.sync_copy(w_hbm.at[pl.ds(base, K)], w_v)
        pltpu.async_copy(g_hbm.at[idx_v], rows_v, sem).wait()

        def _scale(g, _):
            wvec = w_v[pl.ds(g * 16, 16)]
            for l in range(16):
                e = g * 16 + l
                we = wvec[l]
                for j in range(D // 16):
                    sl = pl.ds(j * 16, 16)
                    rows_v[e, sl] = rows_v[e, sl] * we
            return 0
        lax.fori_loop(0, K // 16, _scale, 0)

        pltpu.sync_copy(rows_v, acc_sh.at[dst_v], add=True)
        return 0
    lax.fori_loop(0, CHUNKS, _chunk, 0)

    plsc.subcore_barrier()

    pltpu.sync_copy(acc_sh.at[pl.ds(s * RPS, RPS)],
                    s_out.at[c, pl.ds(s * RPS, RPS)])

    if with_deg:
        plsc.subcore_barrier()
        def _zrow2(i, _):
            for j in range(D // 16):
                rows_v[i, pl.ds(j * 16, 16)] = zero16
            return 0
        lax.fori_loop(0, K, _zrow2, 0)
        for k in range(NZB):
            pltpu.sync_copy(rows_v, acc_sh.at[pl.ds(s * RPS + k * K, K)])
        one0 = jnp.where(lax.iota(jnp.int32, 16) == 0,
                         jnp.float32(1.0), jnp.float32(0.0))
        def _onerow(i, _):
            rows_v[i, pl.ds(0, 16)] = one0
            return 0
        lax.fori_loop(0, K, _onerow, 0)
        plsc.subcore_barrier()

        def _dchunk(t, _):
            base = wid * EPW + t * K
            pltpu.sync_copy(dst_hbm.at[pl.ds(base, K)], dst_v)
            pltpu.sync_copy(rows_v, acc_sh.at[dst_v], add=True)
            return 0
        lax.fori_loop(0, CHUNKS, _dchunk, 0)

        plsc.subcore_barrier()
        pltpu.sync_copy(acc_sh.at[pl.ds(s * RPS, RPS)],
                        deg_out.at[c, pl.ds(s * RPS, RPS)])


def _make_sc_agg(with_deg):
    out_type = [jax.ShapeDtypeStruct((NC, NPAD, D), jnp.float32)]
    if with_deg:
        out_type.append(jax.ShapeDtypeStruct((NC, NPAD, D), jnp.float32))
    scratch = [
        pltpu.VMEM_SHARED((NPAD, D), jnp.float32),
        pltpu.VMEM((K, D), jnp.float32),
        pltpu.VMEM((K,), jnp.int32),
        pltpu.VMEM((K,), jnp.int32),
        pltpu.VMEM((K,), jnp.float32),
        pltpu.SemaphoreType.DMA,
    ]
    mesh = plsc.VectorSubcoreMesh(core_axis_name="c", subcore_axis_name="s")
    return pl.kernel(
        functools.partial(_sc_agg_body, with_deg=with_deg),
        out_type=tuple(out_type),
        mesh=mesh,
        scratch_types=scratch,
        name="sage_sc_agg_deg" if with_deg else "sage_sc_agg",
    )


_sc_agg_deg = _make_sc_agg(True)
_sc_agg = _make_sc_agg(False)


RB = 2000


def _tc1_body(x_ref, ws_ref, wn_ref, b_ref, a_ref, g_ref):
    x = x_ref[...]
    a_ref[...] = jnp.dot(x, ws_ref[...], preferred_element_type=jnp.float32,
                         precision=lax.Precision.HIGHEST) + b_ref[...][None, :]
    g_ref[...] = jnp.dot(x, wn_ref[...], preferred_element_type=jnp.float32,
                         precision=lax.Precision.HIGHEST)


def _tc2_body(a_ref, s0_ref, s1_ref, d0_ref, d1_ref, ws_ref, wn_ref, b_ref,
              a2_ref, g2_ref):
    deg = d0_ref[...][:, :1] + d1_ref[...][:, :1]
    inv = 1.0 / jnp.maximum(deg, 1.0)
    h = a_ref[...] + (s0_ref[...] + s1_ref[...]) * inv
    h = jnp.maximum(h, 0.0)
    a2_ref[...] = jnp.dot(h, ws_ref[...], preferred_element_type=jnp.float32,
                          precision=lax.Precision.HIGHEST) + b_ref[...][None, :]
    g2_ref[...] = jnp.dot(h, wn_ref[...], preferred_element_type=jnp.float32,
                          precision=lax.Precision.HIGHEST)


def _tc3_body(a_ref, s0_ref, s1_ref, d0_ref, d1_ref, o_ref):
    deg = d0_ref[...][:, :1] + d1_ref[...][:, :1]
    inv = 1.0 / jnp.maximum(deg, 1.0)
    o_ref[...] = a_ref[...] + (s0_ref[...] + s1_ref[...]) * inv


def _row_spec(width):
    return pl.BlockSpec((RB, width), lambda i: (i, 0))


def _full_spec(shape):
    return pl.BlockSpec(shape, lambda i: tuple(0 for _ in shape))


_tc1 = pl.pallas_call(
    _tc1_body,
    grid=(N // RB,),
    in_specs=[_row_spec(D), _full_spec((D, D)), _full_spec((D, D)),
              _full_spec((D,))],
    out_specs=[_row_spec(D), _row_spec(D)],
    out_shape=[jax.ShapeDtypeStruct((N, D), jnp.float32)] * 2,
)

_tc2 = pl.pallas_call(
    _tc2_body,
    grid=(N // RB,),
    in_specs=[_row_spec(D), _row_spec(D), _row_spec(D), _row_spec(D),
              _row_spec(D), _full_spec((D, D)), _full_spec((D, D)),
              _full_spec((D,))],
    out_specs=[_row_spec(D), _row_spec(D)],
    out_shape=[jax.ShapeDtypeStruct((N, D), jnp.float32)] * 2,
)

_tc3 = pl.pallas_call(
    _tc3_body,
    grid=(N // RB,),
    in_specs=[_row_spec(D), _row_spec(D), _row_spec(D), _row_spec(D),
              _row_spec(D)],
    out_specs=_row_spec(D),
    out_shape=jax.ShapeDtypeStruct((N, D), jnp.float32),
)


@jax.jit
def kernel(inputs, edge_index, edge_weight, W_self1, W_neigh1, b1,
           W_self2, W_neigh2, b2):
    src = edge_index[0].astype(jnp.int32)
    dst = edge_index[1].astype(jnp.int32)
    w = edge_weight.astype(jnp.float32)

    pad = EPAD - E
    src_p = jnp.concatenate([src, jnp.zeros((pad,), jnp.int32)])
    dst_p = jnp.concatenate([dst, jnp.full((pad,), N, jnp.int32)])
    w_p = jnp.concatenate([w, jnp.zeros((pad,), jnp.float32)])

    a1, g1 = _tc1(inputs, W_self1, W_neigh1, b1)
    s1, degp = _sc_agg_deg(g1, src_p, dst_p, w_p)
    a2, g2 = _tc2(a1, s1[0, :N], s1[1, :N], degp[0, :N], degp[1, :N],
                  W_self2, W_neigh2, b2)
    (s2,) = _sc_agg(g2, src_p, dst_p, w_p)
    return _tc3(a2, s2[0, :N], s2[1, :N], degp[0, :N], degp[1, :N])


# speedup vs baseline: 3.6435x; 1.1272x over previous
import functools

import jax
import jax.numpy as jnp
from jax import lax
from jax.experimental import pallas as pl
from jax.experimental.pallas import tpu as pltpu
from jax.experimental.pallas import tpu_sc as plsc

N = 10000
E = 320000
D = 128

NC = 2
NS = 16
NW = NC * NS
K = 128
CHUNKS = 80
EPW = CHUNKS * K
EPAD = EPW * NW

HC = CHUNKS // 2
NPAD = 10240
RPS = NPAD // NS
NZB = RPS // K


def _sc_agg_body(g_hbm, src_hbm, dst_hbm, w_hbm, *refs, with_deg):
    if with_deg:
        (s_out, deg_out, acc_sh, buf0, buf1, src_t, dst_t, w_t,
         semA, semB) = refs
    else:
        (s_out, acc_sh, buf0, buf1, src_t, dst_t, w_t, semA, semB) = refs

    c = lax.axis_index("c")
    s = lax.axis_index("s")
    wid = c * NS + s

    zero16 = jnp.zeros((16,), jnp.float32)

    def _zrow(i, _):
        for j in range(D // 16):
            buf0[i, pl.ds(j * 16, 16)] = zero16
        return 0
    lax.fori_loop(0, K, _zrow, 0)
    for k in range(NZB):
        pltpu.sync_copy(buf0, acc_sh.at[pl.ds(s * RPS + k * K, K)])

    plsc.subcore_barrier()

    def _scale(buf, t):
        def _grp(g, _):
            wvec = w_t[t, pl.ds(g * 16, 16)]
            for l in range(16):
                e = g * 16 + l
                we = wvec[l]
                for j in range(D // 16):
                    sl = pl.ds(j * 16, 16)
                    buf[e, sl] = buf[e, sl] * we
            return 0
        lax.fori_loop(0, K // 16, _grp, 0)

    for half in range(2):
        pltpu.sync_copy(src_hbm.at[wid, pl.ds(half * HC, HC)], src_t)
        pltpu.sync_copy(dst_hbm.at[wid, pl.ds(half * HC, HC)], dst_t)
        pltpu.sync_copy(w_hbm.at[wid, pl.ds(half * HC, HC)], w_t)
        pltpu.async_copy(g_hbm.at[src_t.at[0]], buf0, semA)

        def _pair(t2, _):
            a = 2 * t2
            b = a + 1
            pltpu.async_copy(g_hbm.at[src_t.at[b]], buf1, semB)
            pltpu.make_async_copy(g_hbm.at[src_t.at[a]], buf0, semA).wait()
            _scale(buf0, a)
            pltpu.sync_copy(buf0, acc_sh.at[dst_t.at[a]], add=True)
            nxt = jnp.minimum(a + 2, HC - 1)
            pltpu.async_copy(g_hbm.at[src_t.at[nxt]], buf0, semA)
            pltpu.make_async_copy(g_hbm.at[src_t.at[b]], buf1, semB).wait()
            _scale(buf1, b)
            pltpu.sync_copy(buf1, acc_sh.at[dst_t.at[b]], add=True)
            return 0
        lax.fori_loop(0, HC // 2, _pair, 0)
        pltpu.make_async_copy(g_hbm.at[src_t.at[0]], buf0, semA).wait()

    plsc.subcore_barrier()

    pltpu.sync_copy(acc_sh.at[pl.ds(s * RPS, RPS)],
                    s_out.at[c, pl.ds(s * RPS, RPS)])

    if with_deg:
        plsc.subcore_barrier()
        def _zrow2(i, _):
            for j in range(D // 16):
                buf0[i, pl.ds(j * 16, 16)] = zero16
            return 0
        lax.fori_loop(0, K, _zrow2, 0)
        for k in range(NZB):
            pltpu.sync_copy(buf0, acc_sh.at[pl.ds(s * RPS + k * K, K)])
        one0 = jnp.where(lax.iota(jnp.int32, 16) == 0,
                         jnp.float32(1.0), jnp.float32(0.0))
        def _onerow(i, _):
            buf0[i, pl.ds(0, 16)] = one0
            return 0
        lax.fori_loop(0, K, _onerow, 0)
        plsc.subcore_barrier()

        for half in range(2):
            pltpu.sync_copy(dst_hbm.at[wid, pl.ds(half * HC, HC)], dst_t)
            def _dchunk(t, _):
                pltpu.sync_copy(buf0, acc_sh.at[dst_t.at[t]], add=True)
                return 0
            lax.fori_loop(0, HC, _dchunk, 0)

        plsc.subcore_barrier()
        pltpu.sync_copy(acc_sh.at[pl.ds(s * RPS, RPS)],
                        deg_out.at[c, pl.ds(s * RPS, RPS)])


def _make_sc_agg(with_deg):
    out_type = [jax.ShapeDtypeStruct((NC, NPAD, D), jnp.float32)]
    if with_deg:
        out_type.append(jax.ShapeDtypeStruct((NC, NPAD, D), jnp.float32))
    scratch = [
        pltpu.VMEM_SHARED((NPAD, D), jnp.float32),
        pltpu.VMEM((K, D), jnp.float32),
        pltpu.VMEM((K, D), jnp.float32),
        pltpu.VMEM((HC, K), jnp.int32),
        pltpu.VMEM((HC, K), jnp.int32),
        pltpu.VMEM((HC, K), jnp.float32),
        pltpu.SemaphoreType.DMA,
        pltpu.SemaphoreType.DMA,
    ]
    mesh = plsc.VectorSubcoreMesh(core_axis_name="c", subcore_axis_name="s")
    return pl.kernel(
        functools.partial(_sc_agg_body, with_deg=with_deg),
        out_type=tuple(out_type),
        mesh=mesh,
        scratch_types=scratch,
        name="sage_sc_agg_deg" if with_deg else "sage_sc_agg",
    )


_sc_agg_deg = _make_sc_agg(True)
_sc_agg = _make_sc_agg(False)


RB = 2000


def _tc1_body(x_ref, ws_ref, wn_ref, b_ref, a_ref, g_ref):
    x = x_ref[...]
    a_ref[...] = jnp.dot(x, ws_ref[...], preferred_element_type=jnp.float32,
                         precision=lax.Precision.HIGHEST) + b_ref[...][None, :]
    g_ref[...] = jnp.dot(x, wn_ref[...], preferred_element_type=jnp.float32,
                         precision=lax.Precision.HIGHEST)


def _tc2_body(a_ref, s0_ref, s1_ref, d0_ref, d1_ref, ws_ref, wn_ref, b_ref,
              a2_ref, g2_ref):
    deg = d0_ref[...][:, :1] + d1_ref[...][:, :1]
    inv = 1.0 / jnp.maximum(deg, 1.0)
    h = a_ref[...] + (s0_ref[...] + s1_ref[...]) * inv
    h = jnp.maximum(h, 0.0)
    a2_ref[...] = jnp.dot(h, ws_ref[...], preferred_element_type=jnp.float32,
                          precision=lax.Precision.HIGHEST) + b_ref[...][None, :]
    g2_ref[...] = jnp.dot(h, wn_ref[...], preferred_element_type=jnp.float32,
                          precision=lax.Precision.HIGHEST)


def _tc3_body(a_ref, s0_ref, s1_ref, d0_ref, d1_ref, o_ref):
    deg = d0_ref[...][:, :1] + d1_ref[...][:, :1]
    inv = 1.0 / jnp.maximum(deg, 1.0)
    o_ref[...] = a_ref[...] + (s0_ref[...] + s1_ref[...]) * inv


def _row_spec(width):
    return pl.BlockSpec((RB, width), lambda i: (i, 0))


def _full_spec(shape):
    return pl.BlockSpec(shape, lambda i: tuple(0 for _ in shape))


_tc1 = pl.pallas_call(
    _tc1_body,
    grid=(N // RB,),
    in_specs=[_row_spec(D), _full_spec((D, D)), _full_spec((D, D)),
              _full_spec((D,))],
    out_specs=[_row_spec(D), _row_spec(D)],
    out_shape=[jax.ShapeDtypeStruct((N, D), jnp.float32)] * 2,
)

_tc2 = pl.pallas_call(
    _tc2_body,
    grid=(N // RB,),
    in_specs=[_row_spec(D), _row_spec(D), _row_spec(D), _row_spec(D),
              _row_spec(D), _full_spec((D, D)), _full_spec((D, D)),
              _full_spec((D,))],
    out_specs=[_row_spec(D), _row_spec(D)],
    out_shape=[jax.ShapeDtypeStruct((N, D), jnp.float32)] * 2,
)

_tc3 = pl.pallas_call(
    _tc3_body,
    grid=(N // RB,),
    in_specs=[_row_spec(D), _row_spec(D), _row_spec(D), _row_spec(D),
              _row_spec(D)],
    out_specs=_row_spec(D),
    out_shape=jax.ShapeDtypeStruct((N, D), jnp.float32),
)


@jax.jit
def kernel(inputs, edge_index, edge_weight, W_self1, W_neigh1, b1,
           W_self2, W_neigh2, b2):
    src = edge_index[0].astype(jnp.int32)
    dst = edge_index[1].astype(jnp.int32)
    w = edge_weight.astype(jnp.float32)

    pad = EPAD - E
    src_p = jnp.concatenate([src, jnp.zeros((pad,), jnp.int32)])
    src_p = src_p.reshape(NW, CHUNKS, K)
    dst_p = jnp.concatenate([dst, jnp.full((pad,), N, jnp.int32)])
    dst_p = dst_p.reshape(NW, CHUNKS, K)
    w_p = jnp.concatenate([w, jnp.zeros((pad,), jnp.float32)])
    w_p = w_p.reshape(NW, CHUNKS, K)

    a1, g1 = _tc1(inputs, W_self1, W_neigh1, b1)
    s1, degp = _sc_agg_deg(g1, src_p, dst_p, w_p)
    a2, g2 = _tc2(a1, s1[0, :N], s1[1, :N], degp[0, :N], degp[1, :N],
                  W_self2, W_neigh2, b2)
    (s2,) = _sc_agg(g2, src_p, dst_p, w_p)
    return _tc3(a2, s2[0, :N], s2[1, :N], degp[0, :N], degp[1, :N])
